# baseline (speedup 1.0000x reference)
import jax
import jax.numpy as jnp
from jax.experimental import pallas as pl
from jax.experimental.pallas import tpu as pltpu

_ROWS = 10000
_FEAT = 256
_BLOCK_ROWS = 10000


def _gelu_exact(x):
    return 0.5 * x * (1.0 + jax.lax.erf(x * 0.7071067811865476))


def _double_gelu_kernel(x_ref, o_ref):
    o_ref[...] = _gelu_exact(_gelu_exact(x_ref[...]))


_GRID = 4


def kernel(embs, edge_index, batch_size):
    del edge_index, batch_size
    n, d = embs.shape
    total = n * d
    lanes = 128
    rows = total // lanes
    block_rows = rows // _GRID
    if total % lanes != 0 or rows % _GRID != 0 or block_rows % 8 != 0:
        rows, lanes, block_rows = n, d, n
    x = embs.reshape(rows, lanes)
    out = pl.pallas_call(
        _double_gelu_kernel,
        grid=(rows // block_rows,),
        in_specs=[pl.BlockSpec((block_rows, lanes), lambda i: (i, 0))],
        out_specs=pl.BlockSpec((block_rows, lanes), lambda i: (i, 0)),
        out_shape=jax.ShapeDtypeStruct((rows, lanes), embs.dtype),
        compiler_params=pltpu.CompilerParams(
            dimension_semantics=("parallel",),
        ),
    )(x)
    return out.reshape(n, d)


# speedup vs baseline: 3.4341x; 3.4341x over previous
import jax
import jax.numpy as jnp
from jax.experimental import pallas as pl
from jax.experimental.pallas import tpu as pltpu

_ROWS = 10000
_FEAT = 256
_BLOCK_ROWS = 10000


def _gelu_exact(x):
    return 0.5 * x * (1.0 + jax.lax.erf(x * 0.7071067811865476))


def _double_gelu_kernel(x_ref, o_ref):
    o_ref[...] = _gelu_exact(_gelu_exact(x_ref[...]))


def kernel(embs, edge_index, batch_size):
    del edge_index, batch_size
    n, d = embs.shape
    block_rows = n // 2 if (n % 2 == 0 and (n // 2) % 8 == 0) else n
    block_cols = d // 2 if d % 256 == 0 else d
    grid = (n // block_rows, d // block_cols)
    return pl.pallas_call(
        _double_gelu_kernel,
        grid=grid,
        in_specs=[pl.BlockSpec((block_rows, block_cols), lambda i, j: (i, j))],
        out_specs=pl.BlockSpec((block_rows, block_cols), lambda i, j: (i, j)),
        out_shape=jax.ShapeDtypeStruct((n, d), embs.dtype),
        compiler_params=pltpu.CompilerParams(
            dimension_semantics=("parallel", "parallel"),
        ),
    )(embs)


# speedup vs baseline: 3.9757x; 1.1577x over previous
import jax
import jax.numpy as jnp
from jax.experimental import pallas as pl
from jax.experimental.pallas import tpu as pltpu

_ROWS = 10000
_FEAT = 256
_BLOCK_ROWS = 10000


def _gelu_exact(x):
    return 0.5 * x * (1.0 + jax.lax.erf(x * 0.7071067811865476))


def _double_gelu_kernel(x_ref, o_ref):
    o_ref[...] = _gelu_exact(_gelu_exact(x_ref[...]))


def kernel(embs, edge_index, batch_size):
    del edge_index, batch_size
    n, d = embs.shape
    block_rows = n // 2 if (n % 2 == 0 and (n // 2) % 8 == 0) else n
    grid = (n // block_rows,)
    return pl.pallas_call(
        _double_gelu_kernel,
        grid=grid,
        in_specs=[pl.BlockSpec((block_rows, d), lambda i: (i, 0))],
        out_specs=pl.BlockSpec((block_rows, d), lambda i: (i, 0)),
        out_shape=jax.ShapeDtypeStruct((n, d), embs.dtype),
        compiler_params=pltpu.CompilerParams(
            dimension_semantics=("parallel",),
        ),
    )(embs)
